# Initial kernel scaffold; baseline (speedup 1.0000x reference)
import jax
import jax.numpy as jnp
from jax.experimental import pallas as pl


def kernel(X, H, Dv_inv_sqrt, De_inv):
    raise NotImplementedError("write your pallas kernel here")


# speedup vs baseline: 1.0129x; 1.0129x over previous
import functools

import jax
import jax.numpy as jnp
from jax.experimental import pallas as pl
from jax.experimental.pallas import tpu as pltpu

N = 10000
M = 4096
D = 128
TM = 256


def _body(x_ref, h_ref, dv_ref, de_ref, o_ref, xn_ref):
    j = pl.program_id(0)

    @pl.when(j == 0)
    def _init():
        xn_ref[...] = dv_ref[...] * x_ref[...]
        o_ref[...] = jnp.zeros_like(o_ref)

    h = h_ref[...]
    xe = jax.lax.dot_general(
        h, xn_ref[...], (((0,), (0,)), ((), ())),
        preferred_element_type=jnp.float32)
    xe = de_ref[...] * xe
    o_ref[...] += jnp.dot(h, xe, preferred_element_type=jnp.float32)

    @pl.when(j == pl.num_programs(0) - 1)
    def _finish():
        o_ref[...] = dv_ref[...] * o_ref[...]


@functools.partial(jax.jit, static_argnames=())
def kernel(X, H, Dv_inv_sqrt, De_inv):
    dv = Dv_inv_sqrt.reshape(N, 1)
    de = De_inv.reshape(M, 1)
    grid = (M // TM,)
    return pl.pallas_call(
        _body,
        grid=grid,
        in_specs=[
            pl.BlockSpec((N, D), lambda j: (0, 0)),
            pl.BlockSpec((N, TM), lambda j: (0, j)),
            pl.BlockSpec((N, 1), lambda j: (0, 0)),
            pl.BlockSpec((TM, 1), lambda j: (j, 0)),
        ],
        out_specs=pl.BlockSpec((N, D), lambda j: (0, 0)),
        out_shape=jax.ShapeDtypeStruct((N, D), jnp.float32),
        scratch_shapes=[pltpu.VMEM((N, D), jnp.float32)],
    )(X, H, dv, de)


# speedup vs baseline: 1.3897x; 1.3720x over previous
import functools

import jax
import jax.numpy as jnp
from jax.experimental import pallas as pl
from jax.experimental.pallas import tpu as pltpu

N = 10000
M = 4096
D = 128
TM = 256


def _body(x_ref, h_ref, dv_ref, de_ref, o_ref, xn_ref):
    j = pl.program_id(0)

    @pl.when(j == 0)
    def _init():
        xn_ref[...] = (dv_ref[...] * x_ref[...]).astype(jnp.bfloat16)
        o_ref[...] = jnp.zeros_like(o_ref)

    h = h_ref[...].astype(jnp.bfloat16)
    xe = jax.lax.dot_general(
        h, xn_ref[...], (((0,), (0,)), ((), ())),
        preferred_element_type=jnp.float32)
    xe = (de_ref[...] * xe).astype(jnp.bfloat16)
    o_ref[...] += jnp.dot(h, xe, preferred_element_type=jnp.float32)

    @pl.when(j == pl.num_programs(0) - 1)
    def _finish():
        o_ref[...] = dv_ref[...] * o_ref[...]


@functools.partial(jax.jit, static_argnames=())
def kernel(X, H, Dv_inv_sqrt, De_inv):
    dv = Dv_inv_sqrt.reshape(N, 1)
    de = De_inv.reshape(M, 1)
    grid = (M // TM,)
    return pl.pallas_call(
        _body,
        grid=grid,
        in_specs=[
            pl.BlockSpec((N, D), lambda j: (0, 0)),
            pl.BlockSpec((N, TM), lambda j: (0, j)),
            pl.BlockSpec((N, 1), lambda j: (0, 0)),
            pl.BlockSpec((TM, 1), lambda j: (j, 0)),
        ],
        out_specs=pl.BlockSpec((N, D), lambda j: (0, 0)),
        out_shape=jax.ShapeDtypeStruct((N, D), jnp.float32),
        scratch_shapes=[pltpu.VMEM((N, D), jnp.bfloat16)],
    )(X, H, dv, de)


# speedup vs baseline: 1.3961x; 1.0046x over previous
import functools

import jax
import jax.numpy as jnp
from jax.experimental import pallas as pl
from jax.experimental.pallas import tpu as pltpu

N = 10000
M = 4096
D = 128
TM = 256


def _body(x_ref, h_ref, dv_ref, de_ref, o_ref, xnt_ref):
    j = pl.program_id(0)

    @pl.when(j == 0)
    def _init():
        xnt_ref[...] = (dv_ref[...] * x_ref[...]).astype(jnp.bfloat16).T
        o_ref[...] = jnp.zeros_like(o_ref)

    h = h_ref[...].astype(jnp.bfloat16)
    xet = jax.lax.dot_general(
        xnt_ref[...], h, (((1,), (0,)), ((), ())),
        preferred_element_type=jnp.float32)
    xet = (de_ref[...] * xet).astype(jnp.bfloat16)
    o_ref[...] += jax.lax.dot_general(
        h, xet, (((1,), (1,)), ((), ())),
        preferred_element_type=jnp.float32)

    @pl.when(j == pl.num_programs(0) - 1)
    def _finish():
        o_ref[...] = dv_ref[...] * o_ref[...]


@functools.partial(jax.jit, static_argnames=())
def kernel(X, H, Dv_inv_sqrt, De_inv):
    dv = Dv_inv_sqrt.reshape(N, 1)
    de = De_inv.reshape(1, M)
    grid = (M // TM,)
    return pl.pallas_call(
        _body,
        grid=grid,
        in_specs=[
            pl.BlockSpec((N, D), lambda j: (0, 0)),
            pl.BlockSpec((N, TM), lambda j: (0, j)),
            pl.BlockSpec((N, 1), lambda j: (0, 0)),
            pl.BlockSpec((1, TM), lambda j: (0, j)),
        ],
        out_specs=pl.BlockSpec((N, D), lambda j: (0, 0)),
        out_shape=jax.ShapeDtypeStruct((N, D), jnp.float32),
        scratch_shapes=[pltpu.VMEM((D, N), jnp.bfloat16)],
    )(X, H, dv, de)
